# baseline (speedup 1.0000x reference)
import functools

import jax
import jax.numpy as jnp
from jax import lax
from jax.experimental import pallas as pl
from jax.experimental.pallas import tpu as pltpu

_B, _L, _K = 64, 2048, 256
_THR = 0.5
_BIG = 1 << 30


_PRE = 8


def _wta_body(x_ref, idx_ref, w_ref, y_ref, idx_s):
    x = x_ref[0]
    kk1 = lax.broadcasted_iota(jnp.int32, (1, _K), 1)
    xp = x[0:_PRE, :]
    iip = lax.broadcasted_iota(jnp.int32, (_PRE, _K), 0)
    kkp = lax.broadcasted_iota(jnp.int32, (_PRE, _K), 1)
    ffp = jnp.min(jnp.where(xp > _THR, iip * _K + kkp, _BIG))

    @pl.when(ffp < _BIG)
    def _():
        idx_s[0] = lax.rem(ffp, _K)

    @pl.when(ffp >= _BIG)
    def _():
        s = x > _THR
        ii = lax.broadcasted_iota(jnp.int32, (_L, _K), 0)
        kk = lax.broadcasted_iota(jnp.int32, (_L, _K), 1)
        ff = jnp.min(jnp.where(s, ii * _K + kk, _BIG))
        total = jnp.sum(x, axis=0, keepdims=True)
        maxv = jnp.max(total)
        k_fb = jnp.min(jnp.where(total == maxv, kk1, _BIG))
        idx_s[0] = jnp.where(ff < _BIG, lax.rem(ff, _K), k_fb)

    idx = idx_s[0]
    w = (kk1 == idx).astype(jnp.float32)
    idx_ref[0] = jnp.full((1, 1), idx, jnp.int32)
    w_ref[0] = w
    y_ref[0] = x * w


def _full_path(spikes):
    idx3, w3, y = pl.pallas_call(
        _wta_body,
        grid=(_B,),
        in_specs=[pl.BlockSpec((1, _L, _K), lambda b: (b, 0, 0))],
        out_specs=[
            pl.BlockSpec((1, 1, 1), lambda b: (b, 0, 0)),
            pl.BlockSpec((1, 1, _K), lambda b: (b, 0, 0)),
            pl.BlockSpec((1, _L, _K), lambda b: (b, 0, 0)),
        ],
        out_shape=[
            jax.ShapeDtypeStruct((_B, 1, 1), jnp.int32),
            jax.ShapeDtypeStruct((_B, 1, _K), jnp.float32),
            jax.ShapeDtypeStruct((_B, _L, _K), jnp.float32),
        ],
        scratch_shapes=[pltpu.SMEM((1,), jnp.int32)],
    )(spikes)
    return idx3[:, 0, 0], w3[:, 0, :], y


def _prefix_body(x_ref, ff_ref):
    x = x_ref[...]
    ii = lax.broadcasted_iota(jnp.int32, (_B, _PRE, _K), 1)
    kk = lax.broadcasted_iota(jnp.int32, (_B, _PRE, _K), 2)
    ff = jnp.min(jnp.where(x > _THR, ii * _K + kk, _BIG), axis=(1, 2))
    ff_ref[...] = ff.reshape(_B, 1, 1)


_GRPS = _K // 128


def _mask_body(idxp_ref, x_ref, idx_ref, w_ref, y_ref):
    b = pl.program_id(0)
    j = pl.program_id(1)
    idx = idxp_ref[b]
    kkl = lax.broadcasted_iota(jnp.int32, (1, 128), 1) + j * 128
    wrow = (kkl == idx).astype(jnp.float32)
    w_ref[0] = wrow
    y_ref[0] = x_ref[0] * wrow

    @pl.when(j == 0)
    def _():
        idx_ref[0] = jnp.full((1, 1), idx, jnp.int32)


def _cheap_path(spikes, idxv):
    grid_spec = pltpu.PrefetchScalarGridSpec(
        num_scalar_prefetch=1,
        grid=(_B, _GRPS),
        in_specs=[
            pl.BlockSpec((1, _L, 128), lambda b, j, idxp: (b, 0, idxp[b] // 128)),
        ],
        out_specs=[
            pl.BlockSpec((1, 1, 1), lambda b, j, idxp: (b, 0, 0)),
            pl.BlockSpec((1, 1, 128), lambda b, j, idxp: (b, 0, j)),
            pl.BlockSpec((1, _L, 128), lambda b, j, idxp: (b, 0, j)),
        ],
    )
    idx3, w3, y = pl.pallas_call(
        _mask_body,
        grid_spec=grid_spec,
        out_shape=[
            jax.ShapeDtypeStruct((_B, 1, 1), jnp.int32),
            jax.ShapeDtypeStruct((_B, 1, _K), jnp.float32),
            jax.ShapeDtypeStruct((_B, _L, _K), jnp.float32),
        ],
    )(idxv, spikes)
    return idx3[:, 0, 0], w3[:, 0, :], y


@jax.jit
def kernel(spikes):
    ff3 = pl.pallas_call(
        _prefix_body,
        grid=(1,),
        in_specs=[pl.BlockSpec((_B, _PRE, _K), lambda i: (0, 0, 0))],
        out_specs=pl.BlockSpec((_B, 1, 1), lambda i: (0, 0, 0)),
        out_shape=jax.ShapeDtypeStruct((_B, 1, 1), jnp.int32),
    )(spikes)
    ff = ff3[:, 0, 0]
    allfound = jnp.all(ff < _BIG)
    idxv = lax.rem(ff, _K)
    return lax.cond(allfound, _cheap_path, lambda x, i: _full_path(x), spikes, idxv)


# speedup vs baseline: 1.6352x; 1.6352x over previous
import functools

import jax
import jax.numpy as jnp
from jax import lax
from jax.experimental import pallas as pl
from jax.experimental.pallas import tpu as pltpu

_B, _L, _K = 64, 2048, 256
_THR = 0.5
_BIG = 1 << 30


_PRE = 8


def _wta_body(x_ref, idx_ref, w_ref, y_ref, idx_s):
    x = x_ref[0]
    kk1 = lax.broadcasted_iota(jnp.int32, (1, _K), 1)
    xp = x[0:_PRE, :]
    iip = lax.broadcasted_iota(jnp.int32, (_PRE, _K), 0)
    kkp = lax.broadcasted_iota(jnp.int32, (_PRE, _K), 1)
    ffp = jnp.min(jnp.where(xp > _THR, iip * _K + kkp, _BIG))

    @pl.when(ffp < _BIG)
    def _():
        idx_s[0] = lax.rem(ffp, _K)

    @pl.when(ffp >= _BIG)
    def _():
        s = x > _THR
        ii = lax.broadcasted_iota(jnp.int32, (_L, _K), 0)
        kk = lax.broadcasted_iota(jnp.int32, (_L, _K), 1)
        ff = jnp.min(jnp.where(s, ii * _K + kk, _BIG))
        total = jnp.sum(x, axis=0, keepdims=True)
        maxv = jnp.max(total)
        k_fb = jnp.min(jnp.where(total == maxv, kk1, _BIG))
        idx_s[0] = jnp.where(ff < _BIG, lax.rem(ff, _K), k_fb)

    idx = idx_s[0]
    w = (kk1 == idx).astype(jnp.float32)
    idx_ref[0] = jnp.full((1, 1), idx, jnp.int32)
    w_ref[0] = w
    y_ref[0] = x * w


def _full_path(spikes):
    idx3, w3, y = pl.pallas_call(
        _wta_body,
        grid=(_B,),
        in_specs=[pl.BlockSpec((1, _L, _K), lambda b: (b, 0, 0))],
        out_specs=[
            pl.BlockSpec((1, 1, 1), lambda b: (b, 0, 0)),
            pl.BlockSpec((1, 1, _K), lambda b: (b, 0, 0)),
            pl.BlockSpec((1, _L, _K), lambda b: (b, 0, 0)),
        ],
        out_shape=[
            jax.ShapeDtypeStruct((_B, 1, 1), jnp.int32),
            jax.ShapeDtypeStruct((_B, 1, _K), jnp.float32),
            jax.ShapeDtypeStruct((_B, _L, _K), jnp.float32),
        ],
        scratch_shapes=[pltpu.SMEM((1,), jnp.int32)],
    )(spikes)
    return idx3[:, 0, 0], w3[:, 0, :], y


def _prefix_body(x_ref, ff_ref):
    x = x_ref[...]
    ii = lax.broadcasted_iota(jnp.int32, (_B, _PRE, _K), 1)
    kk = lax.broadcasted_iota(jnp.int32, (_B, _PRE, _K), 2)
    ff = jnp.min(jnp.where(x > _THR, ii * _K + kk, _BIG), axis=(1, 2))
    ff_ref[...] = ff.reshape(_B, 1, 1)


_GRPS = _K // 128


def _mask_body(idxp_ref, x_ref, idx_ref, w_ref, y_ref):
    b = pl.program_id(0)
    idx = idxp_ref[b]
    grp = idx // 128
    base = pl.multiple_of(grp * 128, 128)
    obase = pl.multiple_of((1 - grp) * 128, 128)
    lanei = lax.broadcasted_iota(jnp.int32, (1, 128), 1)
    wrow = (lanei == idx - base).astype(jnp.float32)
    y_ref[0, :, pl.ds(obase, 128)] = jnp.zeros((_L, 128), jnp.float32)
    y_ref[0, :, pl.ds(base, 128)] = x_ref[0] * wrow
    w_ref[0, :, pl.ds(obase, 128)] = jnp.zeros((1, 128), jnp.float32)
    w_ref[0, :, pl.ds(base, 128)] = wrow
    idx_ref[0] = jnp.full((1, 1), idx, jnp.int32)


def _cheap_path(spikes, idxv):
    grid_spec = pltpu.PrefetchScalarGridSpec(
        num_scalar_prefetch=1,
        grid=(_B,),
        in_specs=[
            pl.BlockSpec((1, _L, 128), lambda b, idxp: (b, 0, idxp[b] // 128)),
        ],
        out_specs=[
            pl.BlockSpec((1, 1, 1), lambda b, idxp: (b, 0, 0)),
            pl.BlockSpec((1, 1, _K), lambda b, idxp: (b, 0, 0)),
            pl.BlockSpec((1, _L, _K), lambda b, idxp: (b, 0, 0)),
        ],
    )
    idx3, w3, y = pl.pallas_call(
        _mask_body,
        grid_spec=grid_spec,
        out_shape=[
            jax.ShapeDtypeStruct((_B, 1, 1), jnp.int32),
            jax.ShapeDtypeStruct((_B, 1, _K), jnp.float32),
            jax.ShapeDtypeStruct((_B, _L, _K), jnp.float32),
        ],
    )(idxv, spikes)
    return idx3[:, 0, 0], w3[:, 0, :], y


@jax.jit
def kernel(spikes):
    ff3 = pl.pallas_call(
        _prefix_body,
        grid=(1,),
        in_specs=[pl.BlockSpec((_B, _PRE, _K), lambda i: (0, 0, 0))],
        out_specs=pl.BlockSpec((_B, 1, 1), lambda i: (0, 0, 0)),
        out_shape=jax.ShapeDtypeStruct((_B, 1, 1), jnp.int32),
    )(spikes)
    ff = ff3[:, 0, 0]
    allfound = jnp.all(ff < _BIG)
    idxv = lax.rem(ff, _K)
    return lax.cond(allfound, _cheap_path, lambda x, i: _full_path(x), spikes, idxv)
